# Initial kernel scaffold; baseline (speedup 1.0000x reference)
import jax
import jax.numpy as jnp
from jax.experimental import pallas as pl


def kernel(feat, embed, edge_index, W1, b1, W2, b2, Wg1, Wg2):
    raise NotImplementedError("write your pallas kernel here")


# speedup vs baseline: 1.7765x; 1.7765x over previous
import functools

import jax
import jax.numpy as jnp
from jax import lax
from jax.experimental import pallas as pl
from jax.experimental.pallas import tpu as pltpu
from jax.experimental.pallas import tpu_sc as plsc

N = 10000
E = 320000
D = 128
H = 64
C = 7

NC = 2
NS = 16
NW = NC * NS

EPW = E // NW
EPC = E // NC
EPW_HALF = EPC // NS
TBL = N * N

_mesh = plsc.VectorSubcoreMesh(core_axis_name="c", subcore_axis_name="s")

def _k1_body(embed_ref, w1a_ref, w1b_ref, b1_ref, g1_ref, g2_ref):
    emb = embed_ref[...]
    g1_ref[...] = jnp.dot(emb, w1a_ref[...], preferred_element_type=jnp.float32) + b1_ref[...][None, :]
    g2_ref[...] = jnp.dot(emb, w1b_ref[...], preferred_element_type=jnp.float32)


K2_CH = 400
K2_TCH = 800


def _k2_body(g1_hbm, g2_hbm, col_hbm, row_hbm, eid_hbm,
             garr1_hbm, garr2_hbm, key_hbm, rev_hbm, tbl_hbm,
             idxv, g1buf, g2buf, colv, rowv, keyv, revv, idv):
    cid = lax.axis_index("c")
    sid = lax.axis_index("s")
    wid = cid * NS + sid

    gbase = wid * EPW

    @pl.loop(0, EPW // K2_CH)
    def _gather(i):
        off = gbase + i * K2_CH
        pltpu.sync_copy(col_hbm.at[pl.ds(off, K2_CH)], idxv)
        pltpu.sync_copy(g1_hbm.at[idxv], g1buf)
        pltpu.sync_copy(g1buf, garr1_hbm.at[pl.ds(off, K2_CH)])
        pltpu.sync_copy(row_hbm.at[pl.ds(off, K2_CH)], idxv)
        pltpu.sync_copy(g2_hbm.at[idxv], g2buf)
        pltpu.sync_copy(g2buf, garr2_hbm.at[pl.ds(off, K2_CH)])

    tbase = sid * (E // NS)

    @pl.when(cid == 0)
    def _table():
        @pl.loop(0, (E // NS) // K2_TCH)
        def _a(i):
            off = tbase + i * K2_TCH
            pltpu.sync_copy(col_hbm.at[pl.ds(off, K2_TCH)], colv)
            pltpu.sync_copy(row_hbm.at[pl.ds(off, K2_TCH)], rowv)

            @pl.loop(0, K2_TCH // 16)
            def _keys(j):
                c16 = colv[pl.ds(j * 16, 16)]
                r16 = rowv[pl.ds(j * 16, 16)]
                keyv[pl.ds(j * 16, 16)] = c16 * N + r16
                revv[pl.ds(j * 16, 16)] = r16 * N + c16

            pltpu.sync_copy(keyv, key_hbm.at[pl.ds(off, K2_TCH)])
            pltpu.sync_copy(revv, rev_hbm.at[pl.ds(off, K2_TCH)])
            pltpu.sync_copy(eid_hbm.at[pl.ds(off, K2_TCH)], idv)
            pltpu.sync_copy(idv, tbl_hbm.at[keyv])


K3_BLK = 8192


def _k3_body(g1_ref, g2_ref, w2_ref, b2_ref, val_ref):
    h = jnp.maximum(g1_ref[...] + g2_ref[...], 0.0)
    s = jnp.sum(h * w2_ref[...][None, :], axis=1) + b2_ref[0]
    val_ref[...] = jax.nn.sigmoid(s)


K4_CH = 2000


def _k4_body(tbl_hbm, key_hbm, rev_hbm, val_hbm, zero_hbm,
             w_hbm, wr_hbm, sums_hbm,
             sumsp, zbuf, keyv, revv, wv, wrv, wrcv, krv, valv):
    cid = lax.axis_index("c")
    sid = lax.axis_index("s")
    zslice = E // NS

    pltpu.sync_copy(zero_hbm.at[pl.ds(sid * zslice, zslice)], zbuf)
    pltpu.sync_copy(zbuf, sumsp.at[pl.ds(sid * zslice, zslice)])
    plsc.subcore_barrier()

    base = cid * EPC + sid * EPW_HALF

    @pl.loop(0, EPW_HALF // K4_CH)
    def _acc(i):
        off = base + i * K4_CH
        pltpu.sync_copy(key_hbm.at[pl.ds(off, K4_CH)], keyv)
        pltpu.sync_copy(rev_hbm.at[pl.ds(off, K4_CH)], revv)
        pltpu.sync_copy(tbl_hbm.at[keyv], wv)
        pltpu.sync_copy(tbl_hbm.at[revv], wrv)

        @pl.loop(0, K4_CH // 16)
        def _clamp(j):
            sl = pl.ds(j * 16, 16)
            wv[sl] = jnp.clip(wv[sl], 0, E - 1)
            wrcv[sl] = jnp.clip(wrv[sl], 0, E - 1)

        pltpu.sync_copy(key_hbm.at[wrcv], krv)

        @pl.loop(0, K4_CH // 16)
        def _validate(j):
            sl = pl.ds(j * 16, 16)
            wr16 = wrv[sl]
            ok = (wr16 >= 0) & (wr16 < E) & (krv[sl] == revv[sl])
            wrv[sl] = jnp.where(ok, wr16, -1)

        pltpu.sync_copy(wv, w_hbm.at[pl.ds(off, K4_CH)])
        pltpu.sync_copy(wrv, wr_hbm.at[pl.ds(off, K4_CH)])
        pltpu.sync_copy(val_hbm.at[pl.ds(off, K4_CH)], valv)
        pltpu.sync_copy(valv, sumsp.at[wv], add=True)

    plsc.subcore_barrier()
    pltpu.sync_copy(sumsp.at[pl.ds(sid * zslice, zslice)], zbuf)
    pltpu.sync_copy(zbuf, sums_hbm.at[pl.ds(cid * E + sid * zslice, zslice)])


K5_CH = 400
K5_HA = 208
K5_HB = 192


def _k5_body(w_hbm, wr_hbm, sums_hbm, col_hbm, row_hbm, feat_hbm,
             zrow_hbm, zmat_hbm,
             em_hbm, agg_hbm, c_hbm,
             aggp, cp, featbuf, wv, wrv, wve, wrcv, wrcev,
             s0w, s1w, s0r, s1r, maskv, cola, rowa, colb, rowb):
    cid = lax.axis_index("c")
    sid = lax.axis_index("s")
    slab = 624

    pltpu.sync_copy(zmat_hbm.at[pl.ds(0, K5_HA)], featbuf)

    @pl.loop(0, 3)
    def _za(k):
        pltpu.sync_copy(featbuf, aggp.at[pl.ds(sid * slab + k * K5_HA, K5_HA)])

    @pl.when(sid == 0)
    def _zc():
        pltpu.sync_copy(featbuf.at[pl.ds(0, 16)], aggp.at[pl.ds(N - 16, 16)])

        @pl.loop(0, N // K5_CH)
        def _zcj(j):
            pltpu.sync_copy(zrow_hbm.at[pl.ds(j * K5_CH, K5_CH)], maskv)
            pltpu.sync_copy(maskv, cp.at[pl.ds(j * K5_CH, K5_CH)])

    plsc.subcore_barrier()

    base = cid * EPC + sid * EPW_HALF

    @pl.loop(0, EPW_HALF // K5_CH)
    def _edges(i):
        off = base + i * K5_CH
        pltpu.sync_copy(w_hbm.at[pl.ds(off, K5_CH)], wv)
        pltpu.sync_copy(wr_hbm.at[pl.ds(off, K5_CH)], wrv)

        @pl.loop(0, K5_CH // 16)
        def _idx(j):
            sl = pl.ds(j * 16, 16)
            w16 = wv[sl]
            wr16 = wrv[sl]
            wrc = jnp.maximum(wr16, 0)
            wve[sl] = w16 + E
            wrcv[sl] = wrc
            wrcev[sl] = wrc + E

        pltpu.sync_copy(sums_hbm.at[wv], s0w)
        pltpu.sync_copy(sums_hbm.at[wve], s1w)
        pltpu.sync_copy(sums_hbm.at[wrcv], s0r)
        pltpu.sync_copy(sums_hbm.at[wrcev], s1r)

        @pl.loop(0, K5_CH // 16)
        def _mask(j):
            sl = pl.ds(j * 16, 16)
            fwd = s0w[sl] + s1w[sl]
            bwd = s0r[sl] + s1r[sl]
            has_rev = wrv[sl] >= 0
            maskv[sl] = 0.5 * (fwd + jnp.where(has_rev, bwd, 0.0))

        pltpu.sync_copy(maskv, em_hbm.at[pl.ds(off, K5_CH)])

        pltpu.sync_copy(col_hbm.at[pl.ds(off, K5_HA)], cola)
        pltpu.sync_copy(row_hbm.at[pl.ds(off, K5_HA)], rowa)
        pltpu.sync_copy(feat_hbm.at[cola], featbuf)

        @pl.loop(0, K5_HA // 16)
        def _scale_a(g):
            mv = maskv[pl.ds(g * 16, 16)]
            for l in range(16):
                m = mv[l]
                for q in range(D // 16):
                    featbuf[g * 16 + l, pl.ds(q * 16, 16)] = (
                        featbuf[g * 16 + l, pl.ds(q * 16, 16)] * m)

        pltpu.sync_copy(featbuf, aggp.at[rowa], add=True)
        pltpu.sync_copy(maskv.at[pl.ds(0, K5_HA)], cp.at[cola], add=True)

        pltpu.sync_copy(col_hbm.at[pl.ds(off + K5_HA, K5_HB)], colb)
        pltpu.sync_copy(row_hbm.at[pl.ds(off + K5_HA, K5_HB)], rowb)
        pltpu.sync_copy(feat_hbm.at[colb], featbuf.at[pl.ds(0, K5_HB)])

        @pl.loop(0, K5_HB // 16)
        def _scale_b(g):
            mv = maskv[pl.ds(K5_HA + g * 16, 16)]
            for l in range(16):
                m = mv[l]
                for q in range(D // 16):
                    featbuf[g * 16 + l, pl.ds(q * 16, 16)] = (
                        featbuf[g * 16 + l, pl.ds(q * 16, 16)] * m)

        pltpu.sync_copy(featbuf.at[pl.ds(0, K5_HB)], aggp.at[rowb], add=True)
        pltpu.sync_copy(maskv.at[pl.ds(K5_HA, K5_HB)], cp.at[colb], add=True)

    plsc.subcore_barrier()

    @pl.loop(0, 3)
    def _da(k):
        pltpu.sync_copy(aggp.at[pl.ds(sid * slab + k * K5_HA, K5_HA)], featbuf)
        pltpu.sync_copy(featbuf,
                        agg_hbm.at[pl.ds(cid * N + sid * slab + k * K5_HA, K5_HA)])

    @pl.when(sid == 0)
    def _dc():
        pltpu.sync_copy(aggp.at[pl.ds(N - 16, 16)], featbuf.at[pl.ds(0, 16)])
        pltpu.sync_copy(featbuf.at[pl.ds(0, 16)],
                        agg_hbm.at[pl.ds(cid * N + N - 16, 16)])

        @pl.loop(0, N // K5_CH)
        def _dcj(j):
            pltpu.sync_copy(cp.at[pl.ds(j * K5_CH, K5_CH)], maskv)
            pltpu.sync_copy(maskv, c_hbm.at[pl.ds(cid * N + j * K5_CH, K5_CH)])


def _k6_body(agg_ref, c_ref, wg1_ref, wg2_ref, probs_ref):
    agg = agg_ref[0:N, :] + agg_ref[N:2 * N, :]
    h1 = jnp.maximum(jnp.dot(agg, wg1_ref[...], preferred_element_type=jnp.float32), 0.0)
    c = c_ref[0:N] + c_ref[N:2 * N]
    s = jnp.sum(c[:, None] * h1, axis=0)
    logits = jnp.dot((s / N)[None, :], wg2_ref[...], preferred_element_type=jnp.float32)
    probs_ref[...] = jax.nn.softmax(logits[0], axis=-1)


@jax.jit
def kernel(feat, embed, edge_index, W1, b1, W2, b2, Wg1, Wg2):
    col = edge_index[0]
    row = edge_index[1]
    W1a = W1[:D, :]
    W1b = W1[D:, :]
    w2row = W2[:, 0]
    eid = jnp.arange(E, dtype=jnp.int32)
    zrow = jnp.zeros((E,), jnp.float32)
    zmat = jnp.zeros((N, D), jnp.float32)

    g1, g2 = pl.pallas_call(
        _k1_body,
        out_shape=(
            jax.ShapeDtypeStruct((N, H), jnp.float32),
            jax.ShapeDtypeStruct((N, H), jnp.float32),
        ),
    )(embed, W1a, W1b, b1)

    k2 = functools.partial(
        pl.kernel,
        compiler_params=pltpu.CompilerParams(use_tc_tiling_on_sc=False),
        out_type=(
            jax.ShapeDtypeStruct((E, H), jnp.float32),
            jax.ShapeDtypeStruct((E, H), jnp.float32),
            jax.ShapeDtypeStruct((E,), jnp.int32),
            jax.ShapeDtypeStruct((E,), jnp.int32),
            jax.ShapeDtypeStruct((TBL,), jnp.int32),
        ),
        mesh=_mesh,
        scratch_types=[
            pltpu.VMEM((K2_CH,), jnp.int32),
            pltpu.VMEM((K2_CH, H), jnp.float32),
            pltpu.VMEM((K2_CH, H), jnp.float32),
            pltpu.VMEM((K2_TCH,), jnp.int32),
            pltpu.VMEM((K2_TCH,), jnp.int32),
            pltpu.VMEM((K2_TCH,), jnp.int32),
            pltpu.VMEM((K2_TCH,), jnp.int32),
            pltpu.VMEM((K2_TCH,), jnp.int32),
        ],
    )(_k2_body)
    garr1, garr2, keys, revs, tbl = k2(g1, g2, col, row, eid)

    values = pl.pallas_call(
        _k3_body,
        grid=(pl.cdiv(E, K3_BLK),),
        in_specs=[
            pl.BlockSpec((K3_BLK, H), lambda i: (i, 0)),
            pl.BlockSpec((K3_BLK, H), lambda i: (i, 0)),
            pl.BlockSpec((H,), lambda i: (0,)),
            pl.BlockSpec((1,), lambda i: (0,)),
        ],
        out_specs=pl.BlockSpec((K3_BLK,), lambda i: (i,)),
        out_shape=jax.ShapeDtypeStruct((E,), jnp.float32),
    )(garr1, garr2, w2row, b2)

    k4 = functools.partial(
        pl.kernel,
        out_type=(
            jax.ShapeDtypeStruct((E,), jnp.int32),
            jax.ShapeDtypeStruct((E,), jnp.int32),
            jax.ShapeDtypeStruct((2 * E,), jnp.float32),
        ),
        mesh=_mesh,
        scratch_types=[
            pltpu.VMEM_SHARED((E,), jnp.float32),
            pltpu.VMEM((E // NS,), jnp.float32),
            pltpu.VMEM((K4_CH,), jnp.int32),
            pltpu.VMEM((K4_CH,), jnp.int32),
            pltpu.VMEM((K4_CH,), jnp.int32),
            pltpu.VMEM((K4_CH,), jnp.int32),
            pltpu.VMEM((K4_CH,), jnp.int32),
            pltpu.VMEM((K4_CH,), jnp.int32),
            pltpu.VMEM((K4_CH,), jnp.float32),
        ],
    )(_k4_body)
    warr, wrarr, sums = k4(tbl, keys, revs, values, zrow)

    k5 = functools.partial(
        pl.kernel,
        out_type=(
            jax.ShapeDtypeStruct((E,), jnp.float32),
            jax.ShapeDtypeStruct((2 * N, D), jnp.float32),
            jax.ShapeDtypeStruct((2 * N,), jnp.float32),
        ),
        mesh=_mesh,
        scratch_types=[
            pltpu.VMEM_SHARED((N, D), jnp.float32),
            pltpu.VMEM_SHARED((N,), jnp.float32),
            pltpu.VMEM((K5_HA, D), jnp.float32),
            pltpu.VMEM((K5_CH,), jnp.int32),
            pltpu.VMEM((K5_CH,), jnp.int32),
            pltpu.VMEM((K5_CH,), jnp.int32),
            pltpu.VMEM((K5_CH,), jnp.int32),
            pltpu.VMEM((K5_CH,), jnp.int32),
            pltpu.VMEM((K5_CH,), jnp.float32),
            pltpu.VMEM((K5_CH,), jnp.float32),
            pltpu.VMEM((K5_CH,), jnp.float32),
            pltpu.VMEM((K5_CH,), jnp.float32),
            pltpu.VMEM((K5_CH,), jnp.float32),
            pltpu.VMEM((K5_HA,), jnp.int32),
            pltpu.VMEM((K5_HA,), jnp.int32),
            pltpu.VMEM((K5_HB,), jnp.int32),
            pltpu.VMEM((K5_HB,), jnp.int32),
        ],
    )(_k5_body)
    edge_mask, aggs, cs = k5(warr, wrarr, sums, col, row, feat, zrow, zmat)

    probs = pl.pallas_call(
        _k6_body,
        out_shape=jax.ShapeDtypeStruct((C,), jnp.float32),
    )(aggs, cs, Wg1, Wg2)

    return probs, edge_mask


# speedup vs baseline: 2.6127x; 1.4707x over previous
import functools

import jax
import jax.numpy as jnp
from jax import lax
from jax.experimental import pallas as pl
from jax.experimental.pallas import tpu as pltpu
from jax.experimental.pallas import tpu_sc as plsc

N = 10000
E = 320000
D = 128
H = 64
C = 7

NC = 2
NS = 16
NW = NC * NS

EPW = E // NW
EPC = E // NC
EPW_HALF = EPC // NS
TBL = N * N

_mesh = plsc.VectorSubcoreMesh(core_axis_name="c", subcore_axis_name="s")

def _k1_body(embed_ref, w1a_ref, w1b_ref, b1_ref, g1_ref, g2_ref):
    emb = embed_ref[...]
    g1_ref[...] = jnp.dot(emb, w1a_ref[...], preferred_element_type=jnp.float32) + b1_ref[...][None, :]
    g2_ref[...] = jnp.dot(emb, w1b_ref[...], preferred_element_type=jnp.float32)


K2_CH = 2000
K2_G = 1000


def _k2_body(g1_hbm, g2_hbm, col_hbm, row_hbm, eid_hbm,
             garr1_hbm, garr2_hbm, key_hbm, rev_hbm, tbl_hbm,
             gbuf, colv, rowv, keyv, revv, idv):
    cid = lax.axis_index("c")
    sid = lax.axis_index("s")
    wid = cid * NS + sid
    gbase = wid * EPW

    @pl.loop(0, EPW // K2_CH)
    def _chunk(i):
        off = gbase + i * K2_CH
        pltpu.sync_copy(col_hbm.at[pl.ds(off, K2_CH)], colv)
        pltpu.sync_copy(row_hbm.at[pl.ds(off, K2_CH)], rowv)
        for h in range(K2_CH // K2_G):
            pltpu.sync_copy(g1_hbm.at[colv.at[pl.ds(h * K2_G, K2_G)]], gbuf)
            pltpu.sync_copy(gbuf, garr1_hbm.at[pl.ds(off + h * K2_G, K2_G)])
            pltpu.sync_copy(g2_hbm.at[rowv.at[pl.ds(h * K2_G, K2_G)]], gbuf)
            pltpu.sync_copy(gbuf, garr2_hbm.at[pl.ds(off + h * K2_G, K2_G)])

        @pl.loop(0, K2_CH // 16)
        def _keys(j):
            c16 = colv[pl.ds(j * 16, 16)]
            r16 = rowv[pl.ds(j * 16, 16)]
            keyv[pl.ds(j * 16, 16)] = c16 * N + r16
            revv[pl.ds(j * 16, 16)] = r16 * N + c16

        pltpu.sync_copy(keyv, key_hbm.at[pl.ds(off, K2_CH)])
        pltpu.sync_copy(revv, rev_hbm.at[pl.ds(off, K2_CH)])
        pltpu.sync_copy(eid_hbm.at[pl.ds(off, K2_CH)], idv)
        pltpu.sync_copy(idv, tbl_hbm.at[keyv])


K3_BLK = 8192


def _k3_body(g1_ref, g2_ref, w2_ref, b2_ref, val_ref):
    h = jnp.maximum(g1_ref[...] + g2_ref[...], 0.0)
    s = jnp.sum(h * w2_ref[...][None, :], axis=1) + b2_ref[0]
    val_ref[...] = jax.nn.sigmoid(s)


K4_CH = 2000


def _k4_body(tbl_hbm, key_hbm, rev_hbm, val_hbm, zero_hbm,
             w_hbm, wr_hbm, sums_hbm,
             sumsp, zbuf, keyv, revv, wv, wrv, wrcv, krv, valv):
    cid = lax.axis_index("c")
    sid = lax.axis_index("s")
    zslice = E // NS

    pltpu.sync_copy(zero_hbm.at[pl.ds(sid * zslice, zslice)], zbuf)
    pltpu.sync_copy(zbuf, sumsp.at[pl.ds(sid * zslice, zslice)])
    plsc.subcore_barrier()

    base = cid * EPC + sid * EPW_HALF

    @pl.loop(0, EPW_HALF // K4_CH)
    def _acc(i):
        off = base + i * K4_CH
        pltpu.sync_copy(key_hbm.at[pl.ds(off, K4_CH)], keyv)
        pltpu.sync_copy(rev_hbm.at[pl.ds(off, K4_CH)], revv)
        pltpu.sync_copy(tbl_hbm.at[keyv], wv)
        pltpu.sync_copy(tbl_hbm.at[revv], wrv)

        @pl.loop(0, K4_CH // 16)
        def _clamp(j):
            sl = pl.ds(j * 16, 16)
            wv[sl] = jnp.clip(wv[sl], 0, E - 1)
            wrcv[sl] = jnp.clip(wrv[sl], 0, E - 1)

        pltpu.sync_copy(key_hbm.at[wrcv], krv)

        @pl.loop(0, K4_CH // 16)
        def _validate(j):
            sl = pl.ds(j * 16, 16)
            wr16 = wrv[sl]
            ok = (wr16 >= 0) & (wr16 < E) & (krv[sl] == revv[sl])
            wrv[sl] = jnp.where(ok, wr16, -1)

        pltpu.sync_copy(wv, w_hbm.at[pl.ds(off, K4_CH)])
        pltpu.sync_copy(wrv, wr_hbm.at[pl.ds(off, K4_CH)])
        pltpu.sync_copy(val_hbm.at[pl.ds(off, K4_CH)], valv)
        pltpu.sync_copy(valv, sumsp.at[wv], add=True)

    plsc.subcore_barrier()
    pltpu.sync_copy(sumsp.at[pl.ds(sid * zslice, zslice)], zbuf)
    pltpu.sync_copy(zbuf, sums_hbm.at[pl.ds(cid * E + sid * zslice, zslice)])


K5_CH = 400
K5_HA = 192
K5_HB = 16


def _k5_body(w_hbm, wr_hbm, sums_hbm, col_hbm, row_hbm, feat_hbm,
             zrow_hbm, zmat_hbm,
             em_hbm, agg_hbm, c_hbm,
             aggp, cp, sumsp, featbuf, wv, wrv, wrcv,
             s0w, s0r, maskv, cola, rowa, colb, rowb):
    cid = lax.axis_index("c")
    sid = lax.axis_index("s")
    slab = 624

    pltpu.sync_copy(zmat_hbm.at[pl.ds(0, K5_HA)], featbuf)

    @pl.loop(0, 3)
    def _za(k):
        pltpu.sync_copy(featbuf, aggp.at[pl.ds(sid * slab + k * K5_HA, K5_HA)])

    pltpu.sync_copy(featbuf.at[pl.ds(0, 48)],
                    aggp.at[pl.ds(sid * slab + 576, 48)])

    @pl.loop(0, (E // NS) // K5_CH)
    def _comb(j):
        o = sid * (E // NS) + j * K5_CH
        pltpu.sync_copy(sums_hbm.at[pl.ds(o, K5_CH)], s0w)
        pltpu.sync_copy(sums_hbm.at[pl.ds(E + o, K5_CH)], s0r)

        @pl.loop(0, K5_CH // 16)
        def _add(t):
            sl = pl.ds(t * 16, 16)
            maskv[sl] = s0w[sl] + s0r[sl]

        pltpu.sync_copy(maskv, sumsp.at[pl.ds(o, K5_CH)])

    @pl.when(sid == 0)
    def _zc():
        pltpu.sync_copy(featbuf.at[pl.ds(0, 16)], aggp.at[pl.ds(N - 16, 16)])

        @pl.loop(0, N // K5_CH)
        def _zcj(j):
            pltpu.sync_copy(zrow_hbm.at[pl.ds(j * K5_CH, K5_CH)], s0w)
            pltpu.sync_copy(s0w, cp.at[pl.ds(j * K5_CH, K5_CH)])

    plsc.subcore_barrier()

    base = cid * EPC + sid * EPW_HALF

    @pl.loop(0, EPW_HALF // K5_CH)
    def _edges(i):
        off = base + i * K5_CH
        pltpu.sync_copy(w_hbm.at[pl.ds(off, K5_CH)], wv)
        pltpu.sync_copy(wr_hbm.at[pl.ds(off, K5_CH)], wrv)

        @pl.loop(0, K5_CH // 16)
        def _idx(j):
            sl = pl.ds(j * 16, 16)
            wrcv[sl] = jnp.maximum(wrv[sl], 0)

        pltpu.sync_copy(sumsp.at[wv], s0w)
        pltpu.sync_copy(sumsp.at[wrcv], s0r)

        @pl.loop(0, K5_CH // 16)
        def _mask(j):
            sl = pl.ds(j * 16, 16)
            has_rev = wrv[sl] >= 0
            maskv[sl] = 0.5 * (s0w[sl] + jnp.where(has_rev, s0r[sl], 0.0))

        pltpu.sync_copy(maskv, em_hbm.at[pl.ds(off, K5_CH)])

        for s in range(2):
            hoff = s * K5_HA
            pltpu.sync_copy(col_hbm.at[pl.ds(off + hoff, K5_HA)], cola)
            pltpu.sync_copy(row_hbm.at[pl.ds(off + hoff, K5_HA)], rowa)
            pltpu.sync_copy(feat_hbm.at[cola], featbuf)

            @pl.loop(0, K5_HA // 16)
            def _scale_a(g):
                mv = maskv[pl.ds(hoff + g * 16, 16)]
                for l in range(16):
                    m = mv[l]
                    for q in range(D // 16):
                        featbuf[g * 16 + l, pl.ds(q * 16, 16)] = (
                            featbuf[g * 16 + l, pl.ds(q * 16, 16)] * m)

            pltpu.sync_copy(featbuf, aggp.at[rowa], add=True)
            pltpu.sync_copy(maskv.at[pl.ds(hoff, K5_HA)], cp.at[cola], add=True)

        toff = 2 * K5_HA
        pltpu.sync_copy(col_hbm.at[pl.ds(off + toff, K5_HB)], colb)
        pltpu.sync_copy(row_hbm.at[pl.ds(off + toff, K5_HB)], rowb)
        pltpu.sync_copy(feat_hbm.at[colb], featbuf.at[pl.ds(0, K5_HB)])

        @pl.loop(0, 1)
        def _scale_b(g):
            mv = maskv[pl.ds(toff, 16)]
            for l in range(16):
                m = mv[l]
                for q in range(D // 16):
                    featbuf[l, pl.ds(q * 16, 16)] = (
                        featbuf[l, pl.ds(q * 16, 16)] * m)

        pltpu.sync_copy(featbuf.at[pl.ds(0, K5_HB)], aggp.at[rowb], add=True)
        pltpu.sync_copy(maskv.at[pl.ds(toff, K5_HB)], cp.at[colb], add=True)

    plsc.subcore_barrier()

    @pl.loop(0, 3)
    def _da(k):
        pltpu.sync_copy(aggp.at[pl.ds(sid * slab + k * K5_HA, K5_HA)], featbuf)
        pltpu.sync_copy(featbuf,
                        agg_hbm.at[pl.ds(cid * N + sid * slab + k * K5_HA, K5_HA)])

    pltpu.sync_copy(aggp.at[pl.ds(sid * slab + 576, 48)], featbuf.at[pl.ds(0, 48)])
    pltpu.sync_copy(featbuf.at[pl.ds(0, 48)],
                    agg_hbm.at[pl.ds(cid * N + sid * slab + 576, 48)])

    @pl.when(sid == 0)
    def _dc():
        pltpu.sync_copy(aggp.at[pl.ds(N - 16, 16)], featbuf.at[pl.ds(0, 16)])
        pltpu.sync_copy(featbuf.at[pl.ds(0, 16)],
                        agg_hbm.at[pl.ds(cid * N + N - 16, 16)])

        @pl.loop(0, N // K5_CH)
        def _dcj(j):
            pltpu.sync_copy(cp.at[pl.ds(j * K5_CH, K5_CH)], maskv)
            pltpu.sync_copy(maskv, c_hbm.at[pl.ds(cid * N + j * K5_CH, K5_CH)])


def _k6_body(agg_ref, c_ref, wg1_ref, wg2_ref, probs_ref):
    agg = agg_ref[0:N, :] + agg_ref[N:2 * N, :]
    h1 = jnp.maximum(jnp.dot(agg, wg1_ref[...], preferred_element_type=jnp.float32), 0.0)
    c = c_ref[0:N] + c_ref[N:2 * N]
    s = jnp.sum(c[:, None] * h1, axis=0)
    logits = jnp.dot((s / N)[None, :], wg2_ref[...], preferred_element_type=jnp.float32)
    probs_ref[...] = jax.nn.softmax(logits[0], axis=-1)


@jax.jit
def kernel(feat, embed, edge_index, W1, b1, W2, b2, Wg1, Wg2):
    col = edge_index[0]
    row = edge_index[1]
    W1a = W1[:D, :]
    W1b = W1[D:, :]
    w2row = W2[:, 0]
    eid = jnp.arange(E, dtype=jnp.int32)
    zrow = jnp.zeros((E,), jnp.float32)
    zmat = jnp.zeros((N, D), jnp.float32)

    g1, g2 = pl.pallas_call(
        _k1_body,
        out_shape=(
            jax.ShapeDtypeStruct((N, H), jnp.float32),
            jax.ShapeDtypeStruct((N, H), jnp.float32),
        ),
    )(embed, W1a, W1b, b1)

    k2 = functools.partial(
        pl.kernel,
        compiler_params=pltpu.CompilerParams(use_tc_tiling_on_sc=False),
        out_type=(
            jax.ShapeDtypeStruct((E, H), jnp.float32),
            jax.ShapeDtypeStruct((E, H), jnp.float32),
            jax.ShapeDtypeStruct((E,), jnp.int32),
            jax.ShapeDtypeStruct((E,), jnp.int32),
            jax.ShapeDtypeStruct((TBL,), jnp.int32),
        ),
        mesh=_mesh,
        scratch_types=[
            pltpu.VMEM((K2_G, H), jnp.float32),
            pltpu.VMEM((K2_CH,), jnp.int32),
            pltpu.VMEM((K2_CH,), jnp.int32),
            pltpu.VMEM((K2_CH,), jnp.int32),
            pltpu.VMEM((K2_CH,), jnp.int32),
            pltpu.VMEM((K2_CH,), jnp.int32),
        ],
    )(_k2_body)
    garr1, garr2, keys, revs, tbl = k2(g1, g2, col, row, eid)

    values = pl.pallas_call(
        _k3_body,
        grid=(pl.cdiv(E, K3_BLK),),
        in_specs=[
            pl.BlockSpec((K3_BLK, H), lambda i: (i, 0)),
            pl.BlockSpec((K3_BLK, H), lambda i: (i, 0)),
            pl.BlockSpec((H,), lambda i: (0,)),
            pl.BlockSpec((1,), lambda i: (0,)),
        ],
        out_specs=pl.BlockSpec((K3_BLK,), lambda i: (i,)),
        out_shape=jax.ShapeDtypeStruct((E,), jnp.float32),
    )(garr1, garr2, w2row, b2)

    k4 = functools.partial(
        pl.kernel,
        out_type=(
            jax.ShapeDtypeStruct((E,), jnp.int32),
            jax.ShapeDtypeStruct((E,), jnp.int32),
            jax.ShapeDtypeStruct((2 * E,), jnp.float32),
        ),
        mesh=_mesh,
        scratch_types=[
            pltpu.VMEM_SHARED((E,), jnp.float32),
            pltpu.VMEM((E // NS,), jnp.float32),
            pltpu.VMEM((K4_CH,), jnp.int32),
            pltpu.VMEM((K4_CH,), jnp.int32),
            pltpu.VMEM((K4_CH,), jnp.int32),
            pltpu.VMEM((K4_CH,), jnp.int32),
            pltpu.VMEM((K4_CH,), jnp.int32),
            pltpu.VMEM((K4_CH,), jnp.int32),
            pltpu.VMEM((K4_CH,), jnp.float32),
        ],
    )(_k4_body)
    warr, wrarr, sums = k4(tbl, keys, revs, values, zrow)

    k5 = functools.partial(
        pl.kernel,
        out_type=(
            jax.ShapeDtypeStruct((E,), jnp.float32),
            jax.ShapeDtypeStruct((2 * N, D), jnp.float32),
            jax.ShapeDtypeStruct((2 * N,), jnp.float32),
        ),
        mesh=_mesh,
        scratch_types=[
            pltpu.VMEM_SHARED((N, D), jnp.float32),
            pltpu.VMEM_SHARED((N,), jnp.float32),
            pltpu.VMEM_SHARED((E,), jnp.float32),
            pltpu.VMEM((K5_HA, D), jnp.float32),
            pltpu.VMEM((K5_CH,), jnp.int32),
            pltpu.VMEM((K5_CH,), jnp.int32),
            pltpu.VMEM((K5_CH,), jnp.int32),
            pltpu.VMEM((K5_CH,), jnp.float32),
            pltpu.VMEM((K5_CH,), jnp.float32),
            pltpu.VMEM((K5_CH,), jnp.float32),
            pltpu.VMEM((K5_HA,), jnp.int32),
            pltpu.VMEM((K5_HA,), jnp.int32),
            pltpu.VMEM((K5_HB,), jnp.int32),
            pltpu.VMEM((K5_HB,), jnp.int32),
        ],
    )(_k5_body)
    edge_mask, aggs, cs = k5(warr, wrarr, sums, col, row, feat, zrow, zmat)

    probs = pl.pallas_call(
        _k6_body,
        out_shape=jax.ShapeDtypeStruct((C,), jnp.float32),
    )(aggs, cs, Wg1, Wg2)

    return probs, edge_mask


# speedup vs baseline: 2.7605x; 1.0566x over previous
import functools

import jax
import jax.numpy as jnp
from jax import lax
from jax.experimental import pallas as pl
from jax.experimental.pallas import tpu as pltpu
from jax.experimental.pallas import tpu_sc as plsc

N = 10000
E = 320000
D = 128
H = 64
C = 7

NC = 2
NS = 16
NW = NC * NS

EPW = E // NW
EPC = E // NC
EPW_HALF = EPC // NS
TBL = N * N

_mesh = plsc.VectorSubcoreMesh(core_axis_name="c", subcore_axis_name="s")

def _k1_body(embed_ref, w1a_ref, w1b_ref, b1_ref, g1_ref, g2_ref):
    emb = embed_ref[...]
    g1_ref[...] = jnp.dot(emb, w1a_ref[...], preferred_element_type=jnp.float32) + b1_ref[...][None, :]
    g2_ref[...] = jnp.dot(emb, w1b_ref[...], preferred_element_type=jnp.float32)


K2_CH = 2000
K2_G = 1000


def _k2_body(g1_hbm, g2_hbm, col_hbm, row_hbm, eid_hbm,
             garr1_hbm, garr2_hbm, key_hbm, rev_hbm, tbl_hbm,
             gbuf, colv, rowv, keyv, revv, idv):
    cid = lax.axis_index("c")
    sid = lax.axis_index("s")
    wid = cid * NS + sid
    gbase = wid * EPW

    @pl.loop(0, EPW // K2_CH)
    def _chunk(i):
        off = gbase + i * K2_CH
        pltpu.sync_copy(col_hbm.at[pl.ds(off, K2_CH)], colv)
        pltpu.sync_copy(row_hbm.at[pl.ds(off, K2_CH)], rowv)
        for h in range(K2_CH // K2_G):
            pltpu.sync_copy(g1_hbm.at[colv.at[pl.ds(h * K2_G, K2_G)]], gbuf)
            pltpu.sync_copy(gbuf, garr1_hbm.at[pl.ds(off + h * K2_G, K2_G)])
            pltpu.sync_copy(g2_hbm.at[rowv.at[pl.ds(h * K2_G, K2_G)]], gbuf)
            pltpu.sync_copy(gbuf, garr2_hbm.at[pl.ds(off + h * K2_G, K2_G)])

        @pl.loop(0, K2_CH // 16)
        def _keys(j):
            c16 = colv[pl.ds(j * 16, 16)]
            r16 = rowv[pl.ds(j * 16, 16)]
            keyv[pl.ds(j * 16, 16)] = c16 * N + r16
            revv[pl.ds(j * 16, 16)] = r16 * N + c16

        pltpu.sync_copy(keyv, key_hbm.at[pl.ds(off, K2_CH)])
        pltpu.sync_copy(revv, rev_hbm.at[pl.ds(off, K2_CH)])
        pltpu.sync_copy(eid_hbm.at[pl.ds(off, K2_CH)], idv)
        pltpu.sync_copy(idv, tbl_hbm.at[keyv])


K3_BLK = 8192


def _k3_body(g1_ref, g2_ref, w2_ref, b2_ref, val_ref):
    h = jnp.maximum(g1_ref[...] + g2_ref[...], 0.0)
    s = jnp.sum(h * w2_ref[...][None, :], axis=1) + b2_ref[0]
    val_ref[...] = jax.nn.sigmoid(s)


K4_CH = 2000


def _k4a_body(tbl_hbm, key_hbm, rev_hbm,
              w_hbm, wr_hbm,
              keyv, revv, wv, wrv, wrcv, krv):
    cid = lax.axis_index("c")
    sid = lax.axis_index("s")
    base = cid * EPC + sid * EPW_HALF

    @pl.loop(0, EPW_HALF // K4_CH)
    def _acc(i):
        off = base + i * K4_CH
        pltpu.sync_copy(key_hbm.at[pl.ds(off, K4_CH)], keyv)
        pltpu.sync_copy(rev_hbm.at[pl.ds(off, K4_CH)], revv)
        pltpu.sync_copy(tbl_hbm.at[keyv], wv)
        pltpu.sync_copy(tbl_hbm.at[revv], wrv)

        @pl.loop(0, K4_CH // 16)
        def _clamp(j):
            sl = pl.ds(j * 16, 16)
            wv[sl] = jnp.clip(wv[sl], 0, E - 1)
            wrcv[sl] = jnp.clip(wrv[sl], 0, E - 1)

        pltpu.sync_copy(key_hbm.at[wrcv], krv)

        @pl.loop(0, K4_CH // 16)
        def _validate(j):
            sl = pl.ds(j * 16, 16)
            wr16 = wrv[sl]
            ok = (wr16 >= 0) & (wr16 < E) & (krv[sl] == revv[sl])
            wrv[sl] = jnp.where(ok, wr16, -1)

        pltpu.sync_copy(wv, w_hbm.at[pl.ds(off, K4_CH)])
        pltpu.sync_copy(wrv, wr_hbm.at[pl.ds(off, K4_CH)])


def _k4b_body(w_hbm, val_hbm, zero_hbm,
              sums_hbm,
              sumsp, zbuf, wv, valv):
    cid = lax.axis_index("c")
    sid = lax.axis_index("s")
    zslice = E // NS

    pltpu.sync_copy(zero_hbm.at[pl.ds(sid * zslice, zslice)], zbuf)
    pltpu.sync_copy(zbuf, sumsp.at[pl.ds(sid * zslice, zslice)])
    plsc.subcore_barrier()

    base = cid * EPC + sid * EPW_HALF

    @pl.loop(0, EPW_HALF // K4_CH)
    def _acc(i):
        off = base + i * K4_CH
        pltpu.sync_copy(w_hbm.at[pl.ds(off, K4_CH)], wv)
        pltpu.sync_copy(val_hbm.at[pl.ds(off, K4_CH)], valv)
        pltpu.sync_copy(valv, sumsp.at[wv], add=True)

    plsc.subcore_barrier()
    pltpu.sync_copy(sumsp.at[pl.ds(sid * zslice, zslice)], zbuf)
    pltpu.sync_copy(zbuf, sums_hbm.at[pl.ds(cid * E + sid * zslice, zslice)])


K5_CH = 400
K5_HA = 192
K5_HB = 16


def _k5_body(w_hbm, wr_hbm, sums_hbm, col_hbm, row_hbm, feat_hbm,
             zrow_hbm, zmat_hbm,
             em_hbm, agg_hbm, c_hbm,
             aggp, cp, sumsp, featbuf, wv, wrv, wrcv,
             s0w, s0r, maskv, cola, rowa, colb, rowb):
    cid = lax.axis_index("c")
    sid = lax.axis_index("s")
    slab = 624

    pltpu.sync_copy(zmat_hbm.at[pl.ds(0, K5_HA)], featbuf)

    @pl.loop(0, 3)
    def _za(k):
        pltpu.sync_copy(featbuf, aggp.at[pl.ds(sid * slab + k * K5_HA, K5_HA)])

    pltpu.sync_copy(featbuf.at[pl.ds(0, 48)],
                    aggp.at[pl.ds(sid * slab + 576, 48)])

    @pl.loop(0, (E // NS) // K5_CH)
    def _comb(j):
        o = sid * (E // NS) + j * K5_CH
        pltpu.sync_copy(sums_hbm.at[pl.ds(o, K5_CH)], s0w)
        pltpu.sync_copy(sums_hbm.at[pl.ds(E + o, K5_CH)], s0r)

        @pl.loop(0, K5_CH // 16)
        def _add(t):
            sl = pl.ds(t * 16, 16)
            maskv[sl] = s0w[sl] + s0r[sl]

        pltpu.sync_copy(maskv, sumsp.at[pl.ds(o, K5_CH)])

    @pl.when(sid == 0)
    def _zc():
        pltpu.sync_copy(featbuf.at[pl.ds(0, 16)], aggp.at[pl.ds(N - 16, 16)])

        @pl.loop(0, N // K5_CH)
        def _zcj(j):
            pltpu.sync_copy(zrow_hbm.at[pl.ds(j * K5_CH, K5_CH)], s0w)
            pltpu.sync_copy(s0w, cp.at[pl.ds(j * K5_CH, K5_CH)])

    plsc.subcore_barrier()

    base = cid * EPC + sid * EPW_HALF

    @pl.loop(0, EPW_HALF // K5_CH)
    def _edges(i):
        off = base + i * K5_CH
        pltpu.sync_copy(w_hbm.at[pl.ds(off, K5_CH)], wv)
        pltpu.sync_copy(wr_hbm.at[pl.ds(off, K5_CH)], wrv)

        @pl.loop(0, K5_CH // 16)
        def _idx(j):
            sl = pl.ds(j * 16, 16)
            wrcv[sl] = jnp.maximum(wrv[sl], 0)

        pltpu.sync_copy(sumsp.at[wv], s0w)
        pltpu.sync_copy(sumsp.at[wrcv], s0r)

        @pl.loop(0, K5_CH // 16)
        def _mask(j):
            sl = pl.ds(j * 16, 16)
            has_rev = wrv[sl] >= 0
            maskv[sl] = 0.5 * (s0w[sl] + jnp.where(has_rev, s0r[sl], 0.0))

        pltpu.sync_copy(maskv, em_hbm.at[pl.ds(off, K5_CH)])

        for s in range(2):
            hoff = s * K5_HA
            pltpu.sync_copy(col_hbm.at[pl.ds(off + hoff, K5_HA)], cola)
            pltpu.sync_copy(row_hbm.at[pl.ds(off + hoff, K5_HA)], rowa)
            pltpu.sync_copy(feat_hbm.at[cola], featbuf)

            @pl.loop(0, K5_HA // 16)
            def _scale_a(g):
                mv = maskv[pl.ds(hoff + g * 16, 16)]
                for l in range(16):
                    m = mv[l]
                    for q in range(D // 16):
                        featbuf[g * 16 + l, pl.ds(q * 16, 16)] = (
                            featbuf[g * 16 + l, pl.ds(q * 16, 16)] * m)

            pltpu.sync_copy(featbuf, aggp.at[rowa], add=True)
            pltpu.sync_copy(maskv.at[pl.ds(hoff, K5_HA)], cp.at[cola], add=True)

        toff = 2 * K5_HA
        pltpu.sync_copy(col_hbm.at[pl.ds(off + toff, K5_HB)], colb)
        pltpu.sync_copy(row_hbm.at[pl.ds(off + toff, K5_HB)], rowb)
        pltpu.sync_copy(feat_hbm.at[colb], featbuf.at[pl.ds(0, K5_HB)])

        @pl.loop(0, 1)
        def _scale_b(g):
            mv = maskv[pl.ds(toff, 16)]
            for l in range(16):
                m = mv[l]
                for q in range(D // 16):
                    featbuf[l, pl.ds(q * 16, 16)] = (
                        featbuf[l, pl.ds(q * 16, 16)] * m)

        pltpu.sync_copy(featbuf.at[pl.ds(0, K5_HB)], aggp.at[rowb], add=True)
        pltpu.sync_copy(maskv.at[pl.ds(toff, K5_HB)], cp.at[colb], add=True)

    plsc.subcore_barrier()

    @pl.loop(0, 3)
    def _da(k):
        pltpu.sync_copy(aggp.at[pl.ds(sid * slab + k * K5_HA, K5_HA)], featbuf)
        pltpu.sync_copy(featbuf,
                        agg_hbm.at[pl.ds(cid * N + sid * slab + k * K5_HA, K5_HA)])

    pltpu.sync_copy(aggp.at[pl.ds(sid * slab + 576, 48)], featbuf.at[pl.ds(0, 48)])
    pltpu.sync_copy(featbuf.at[pl.ds(0, 48)],
                    agg_hbm.at[pl.ds(cid * N + sid * slab + 576, 48)])

    @pl.when(sid == 0)
    def _dc():
        pltpu.sync_copy(aggp.at[pl.ds(N - 16, 16)], featbuf.at[pl.ds(0, 16)])
        pltpu.sync_copy(featbuf.at[pl.ds(0, 16)],
                        agg_hbm.at[pl.ds(cid * N + N - 16, 16)])

        @pl.loop(0, N // K5_CH)
        def _dcj(j):
            pltpu.sync_copy(cp.at[pl.ds(j * K5_CH, K5_CH)], maskv)
            pltpu.sync_copy(maskv, c_hbm.at[pl.ds(cid * N + j * K5_CH, K5_CH)])


def _k6_body(agg_ref, c_ref, wg1_ref, wg2_ref, probs_ref):
    agg = agg_ref[0:N, :] + agg_ref[N:2 * N, :]
    h1 = jnp.maximum(jnp.dot(agg, wg1_ref[...], preferred_element_type=jnp.float32), 0.0)
    c = c_ref[0:N] + c_ref[N:2 * N]
    s = jnp.sum(c[:, None] * h1, axis=0)
    logits = jnp.dot((s / N)[None, :], wg2_ref[...], preferred_element_type=jnp.float32)
    probs_ref[...] = jax.nn.softmax(logits[0], axis=-1)


@jax.jit
def kernel(feat, embed, edge_index, W1, b1, W2, b2, Wg1, Wg2):
    col = edge_index[0]
    row = edge_index[1]
    W1a = W1[:D, :]
    W1b = W1[D:, :]
    w2row = W2[:, 0]
    eid = jnp.arange(E, dtype=jnp.int32)
    zrow = jnp.zeros((E,), jnp.float32)
    zmat = jnp.zeros((N, D), jnp.float32)

    g1, g2 = pl.pallas_call(
        _k1_body,
        out_shape=(
            jax.ShapeDtypeStruct((N, H), jnp.float32),
            jax.ShapeDtypeStruct((N, H), jnp.float32),
        ),
    )(embed, W1a, W1b, b1)

    k2 = functools.partial(
        pl.kernel,
        compiler_params=pltpu.CompilerParams(use_tc_tiling_on_sc=False),
        out_type=(
            jax.ShapeDtypeStruct((E, H), jnp.float32),
            jax.ShapeDtypeStruct((E, H), jnp.float32),
            jax.ShapeDtypeStruct((E,), jnp.int32),
            jax.ShapeDtypeStruct((E,), jnp.int32),
            jax.ShapeDtypeStruct((TBL,), jnp.int32),
        ),
        mesh=_mesh,
        scratch_types=[
            pltpu.VMEM((K2_G, H), jnp.float32),
            pltpu.VMEM((K2_CH,), jnp.int32),
            pltpu.VMEM((K2_CH,), jnp.int32),
            pltpu.VMEM((K2_CH,), jnp.int32),
            pltpu.VMEM((K2_CH,), jnp.int32),
            pltpu.VMEM((K2_CH,), jnp.int32),
        ],
    )(_k2_body)
    garr1, garr2, keys, revs, tbl = k2(g1, g2, col, row, eid)

    k4a = functools.partial(
        pl.kernel,
        out_type=(
            jax.ShapeDtypeStruct((E,), jnp.int32),
            jax.ShapeDtypeStruct((E,), jnp.int32),
        ),
        mesh=_mesh,
        scratch_types=[
            pltpu.VMEM((K4_CH,), jnp.int32),
            pltpu.VMEM((K4_CH,), jnp.int32),
            pltpu.VMEM((K4_CH,), jnp.int32),
            pltpu.VMEM((K4_CH,), jnp.int32),
            pltpu.VMEM((K4_CH,), jnp.int32),
            pltpu.VMEM((K4_CH,), jnp.int32),
        ],
    )(_k4a_body)
    warr, wrarr = k4a(tbl, keys, revs)

    values = pl.pallas_call(
        _k3_body,
        grid=(pl.cdiv(E, K3_BLK),),
        in_specs=[
            pl.BlockSpec((K3_BLK, H), lambda i: (i, 0)),
            pl.BlockSpec((K3_BLK, H), lambda i: (i, 0)),
            pl.BlockSpec((H,), lambda i: (0,)),
            pl.BlockSpec((1,), lambda i: (0,)),
        ],
        out_specs=pl.BlockSpec((K3_BLK,), lambda i: (i,)),
        out_shape=jax.ShapeDtypeStruct((E,), jnp.float32),
    )(garr1, garr2, w2row, b2)

    k4b = functools.partial(
        pl.kernel,
        out_type=jax.ShapeDtypeStruct((2 * E,), jnp.float32),
        mesh=_mesh,
        scratch_types=[
            pltpu.VMEM_SHARED((E,), jnp.float32),
            pltpu.VMEM((E // NS,), jnp.float32),
            pltpu.VMEM((K4_CH,), jnp.int32),
            pltpu.VMEM((K4_CH,), jnp.float32),
        ],
    )(_k4b_body)
    sums = k4b(warr, values, zrow)

    k5 = functools.partial(
        pl.kernel,
        out_type=(
            jax.ShapeDtypeStruct((E,), jnp.float32),
            jax.ShapeDtypeStruct((2 * N, D), jnp.float32),
            jax.ShapeDtypeStruct((2 * N,), jnp.float32),
        ),
        mesh=_mesh,
        scratch_types=[
            pltpu.VMEM_SHARED((N, D), jnp.float32),
            pltpu.VMEM_SHARED((N,), jnp.float32),
            pltpu.VMEM_SHARED((E,), jnp.float32),
            pltpu.VMEM((K5_HA, D), jnp.float32),
            pltpu.VMEM((K5_CH,), jnp.int32),
            pltpu.VMEM((K5_CH,), jnp.int32),
            pltpu.VMEM((K5_CH,), jnp.int32),
            pltpu.VMEM((K5_CH,), jnp.float32),
            pltpu.VMEM((K5_CH,), jnp.float32),
            pltpu.VMEM((K5_CH,), jnp.float32),
            pltpu.VMEM((K5_HA,), jnp.int32),
            pltpu.VMEM((K5_HA,), jnp.int32),
            pltpu.VMEM((K5_HB,), jnp.int32),
            pltpu.VMEM((K5_HB,), jnp.int32),
        ],
    )(_k5_body)
    edge_mask, aggs, cs = k5(warr, wrarr, sums, col, row, feat, zrow, zmat)

    probs = pl.pallas_call(
        _k6_body,
        out_shape=jax.ShapeDtypeStruct((C,), jnp.float32),
    )(aggs, cs, Wg1, Wg2)

    return probs, edge_mask
